# baseline (speedup 1.0000x reference)
import functools

import jax
import jax.numpy as jnp
from jax import lax
from jax.experimental import pallas as pl
from jax.experimental.pallas import tpu as pltpu
from jax.experimental.pallas import tpu_sc as plsc

N_NODES = 10000
N_PAD = 10240
EMB = 128
N_EDGES = 320000
NC = 2
NS = 16
HALF = N_PAD // NC
TRASH = 128
ACC_ROWS = HALF + TRASH
EDGES_PER_TILE = N_EDGES // NS
CHUNK = 128
N_CHUNKS = 159
EDGES_PER_TILE_PAD = N_CHUNKS * CHUNK
NBUF = 3


def _spmm_sc(x, edata, ew):
  mesh = plsc.VectorSubcoreMesh(core_axis_name="c", subcore_axis_name="s")

  @functools.partial(
      pl.kernel,
      mesh=mesh,
      out_type=jax.ShapeDtypeStruct((N_PAD, EMB), jnp.float32),
      scratch_types=[
          pltpu.VMEM((NBUF, 2, CHUNK), jnp.int32),
          pltpu.VMEM((NBUF, 1, CHUNK), jnp.float32),
          pltpu.VMEM((NBUF, 1, CHUNK), jnp.int32),
          pltpu.VMEM((NBUF, CHUNK, EMB), jnp.float32),
          pltpu.VMEM_SHARED((ACC_ROWS, EMB), jnp.float32),
          pltpu.SemaphoreType.DMA,
          pltpu.SemaphoreType.DMA,
          pltpu.SemaphoreType.DMA,
          pltpu.SemaphoreType.DMA,
          pltpu.SemaphoreType.DMA,
          pltpu.SemaphoreType.DMA,
          pltpu.SemaphoreType.DMA,
          pltpu.SemaphoreType.DMA,
          pltpu.SemaphoreType.DMA,
          pltpu.SemaphoreType.DMA,
          pltpu.SemaphoreType.DMA,
          pltpu.SemaphoreType.DMA,
      ],
  )
  def spmm(x_hbm, ed_hbm, ew_hbm, out_hbm, win_v, wwin_v, dmap_v, rows_v,
           acc_sh, sw0, sw1, sw2, sv0, sv1, sv2, sg0, sg1, sg2,
           ss0, ss1, ss2):
    sem_w = (sw0, sw1, sw2)
    sem_v = (sv0, sv1, sv2)
    sem_g = (sg0, sg1, sg2)
    sem_s = (ss0, ss1, ss2)
    c = lax.axis_index("c")
    s = lax.axis_index("s")
    lo = c * HALF

    z16 = jnp.zeros((16,), jnp.float32)

    def zfill(i, _):
      r = i // (EMB // 16)
      j = i % (EMB // 16)
      rows_v[0, r, pl.ds(j * 16, 16)] = z16
      return 0

    lax.fori_loop(0, CHUNK * (EMB // 16), zfill, 0)
    rows_per_tile = ACC_ROWS // NS
    base = s * rows_per_tile
    pltpu.sync_copy(rows_v.at[0], acc_sh.at[pl.ds(base, CHUNK)])
    pltpu.sync_copy(rows_v.at[0], acc_sh.at[pl.ds(base + CHUNK, CHUNK)])
    pltpu.sync_copy(rows_v.at[0, pl.ds(0, rows_per_tile - 2 * CHUNK)],
                    acc_sh.at[pl.ds(base + 2 * CHUNK,
                                    rows_per_tile - 2 * CHUNK)])
    plsc.subcore_barrier()

    def issue_win(k, b):
      pltpu.async_copy(ed_hbm.at[s, k], win_v.at[b], sem_w[b])
      pltpu.async_copy(ew_hbm.at[s, k], wwin_v.at[b], sem_v[b])

    def wait_win(k, b):
      pltpu.make_async_copy(ed_hbm.at[s, k], win_v.at[b], sem_w[b]).wait()
      pltpu.make_async_copy(ew_hbm.at[s, k], wwin_v.at[b], sem_v[b]).wait()

    def remap(b):
      for g in range(CHUNK // 16):
        sl = pl.ds(g * 16, 16)
        d = win_v[b, 1, sl] - lo
        valid = (d >= 0) & (d < HALF)
        dmap_v[b, 0, sl] = jnp.where(valid, d, HALF)

    def issue_gather(b):
      pltpu.async_copy(x_hbm.at[win_v.at[b, 0]], rows_v.at[b], sem_g[b])

    def wait_gather(b):
      pltpu.make_async_copy(
          x_hbm.at[win_v.at[b, 0]], rows_v.at[b], sem_g[b]).wait()

    def scale(b):
      def group_body(g, _):
        wg = wwin_v[b, 0, pl.ds(g * 16, 16)]
        for e in range(16):
          we = wg[e]
          row = g * 16 + e
          for j in range(EMB // 16):
            sl = pl.ds(j * 16, 16)
            rows_v[b, row, sl] = rows_v[b, row, sl] * we
        return 0

      lax.fori_loop(0, CHUNK // 16, group_body, 0)

    def issue_scatter(b):
      pltpu.async_copy(rows_v.at[b], acc_sh.at[dmap_v.at[b, 0]], sem_s[b],
                       add=True)

    def wait_scatter(b):
      pltpu.make_async_copy(
          rows_v.at[b], acc_sh.at[dmap_v.at[b, 0]], sem_s[b]).wait()

    issue_win(0, 0)
    wait_win(0, 0)
    remap(0)
    issue_gather(0)
    issue_win(1, 1)

    def iter_body(i, _):
      k0 = i * NBUF
      for b in range(NBUF):
        k = k0 + b
        b1 = (b + 1) % NBUF
        b2 = (b + 2) % NBUF

        @pl.when(k >= 2)
        def _():
          wait_scatter(b1)

        @pl.when(k + 1 < N_CHUNKS)
        def _():
          wait_win(k + 1, b1)
          remap(b1)
          issue_gather(b1)

        @pl.when(k + 2 < N_CHUNKS)
        def _():
          issue_win(k + 2, b2)

        wait_gather(b)
        scale(b)
        issue_scatter(b)
      return 0

    lax.fori_loop(0, N_CHUNKS // NBUF, iter_body, 0)
    wait_scatter((N_CHUNKS - 2) % NBUF)
    wait_scatter((N_CHUNKS - 1) % NBUF)
    plsc.subcore_barrier()

    out_rows = HALF // NS
    pltpu.sync_copy(acc_sh.at[pl.ds(s * out_rows, out_rows)],
                    out_hbm.at[pl.ds(c * HALF + s * out_rows, out_rows)])

  return spmm(x, edata, ew)


_ROWS_BLK = 1024


def _finalize(bvec, x0, x1, x2):
  def body(b_ref, x0_ref, x1_ref, x2_ref, o_ref):
    def n(v):
      ss = jnp.sum(v * v, axis=-1, keepdims=True)
      nrm = jnp.sqrt(ss)
      return v / jnp.maximum(nrm, 1e-12)

    acc = (b_ref[0] * n(x0_ref[...]) + b_ref[1] * n(x1_ref[...])
           + b_ref[2] * n(x2_ref[...]))
    o_ref[...] = 2.0 * acc

  blk = lambda: pl.BlockSpec((_ROWS_BLK, EMB), lambda i: (i, 0))
  return pl.pallas_call(
      body,
      grid=(N_PAD // _ROWS_BLK,),
      in_specs=[pl.BlockSpec(memory_space=pltpu.SMEM), blk(), blk(), blk()],
      out_specs=blk(),
      out_shape=jax.ShapeDtypeStruct((N_PAD, EMB), jnp.float32),
  )(bvec, x0, x1, x2)


def kernel(nodes_emb, edge_weight, b, edge_index):
  def tile_pad(a, fill):
    a = a.reshape(NS, EDGES_PER_TILE)
    a = jnp.pad(a, ((0, 0), (0, EDGES_PER_TILE_PAD - EDGES_PER_TILE)),
                constant_values=fill)
    return a.reshape(NS, N_CHUNKS, 1, CHUNK)

  src_t = tile_pad(edge_index[0], 0)
  dst_t = tile_pad(edge_index[1], N_PAD)
  edata = jnp.concatenate([src_t, dst_t], axis=2)
  ew = tile_pad(edge_weight, 0.0)
  bvec = b.reshape(3)

  x0 = jnp.pad(nodes_emb, ((0, N_PAD - N_NODES), (0, 0)))
  x1 = _spmm_sc(x0, edata, ew)
  x2 = _spmm_sc(x1, edata, ew)
  core = _finalize(bvec, x0, x1, x2)
  zeros = jnp.zeros((1, EMB), jnp.float32)
  return jnp.concatenate([zeros, core[:N_NODES]], axis=0)
